# baseline (speedup 1.0000x reference)
import functools

import jax
import jax.numpy as jnp
from jax import lax
from jax.experimental import pallas as pl
from jax.experimental.pallas import tpu as pltpu
from jax.experimental.pallas import tpu_sc as plsc

NC = 2
NS = 16
LANES = 16
NW = NC * NS
BATCH = 16384
D = 64
BPW = BATCH // NW
CHUNK = 128
NCH = BPW // CHUNK

_mesh = plsc.VectorSubcoreMesh(
    core_axis_name="c", subcore_axis_name="s", num_cores=NC, num_subcores=NS
)


@functools.partial(
    pl.kernel,
    out_type=jax.ShapeDtypeStruct((BATCH,), jnp.float32),
    mesh=_mesh,
    scratch_types=[
        pltpu.VMEM((NCH, CHUNK), jnp.int32),
        pltpu.VMEM((NCH, CHUNK), jnp.int32),
        pltpu.VMEM((BPW, D), jnp.float32),
        pltpu.VMEM((BPW, D), jnp.float32),
        pltpu.VMEM((144,), jnp.float32),
        pltpu.VMEM((BPW,), jnp.float32),
        pltpu.VMEM((LANES, LANES), jnp.float32),
        pltpu.SemaphoreType.DMA,
        pltpu.SemaphoreType.DMA,
    ],
    compiler_params=pltpu.CompilerParams(
        needs_layout_passes=False, use_tc_tiling_on_sc=False),
)
def _rec_kernel(uid_hbm, iid_hbm, utab_hbm, itab_hbm, wb_hbm, out_hbm,
                idx_u, idx_i, rows_u, rows_i, w_v, out_v, tr_v, sem_u, sem_i):
    wid = lax.axis_index("s") * NC + lax.axis_index("c")
    base = wid * BPW

    pltpu.sync_copy(uid_hbm.at[pl.ds(wid * NCH, NCH)], idx_u)
    pltpu.sync_copy(iid_hbm.at[pl.ds(wid * NCH, NCH)], idx_i)
    pltpu.sync_copy(wb_hbm, w_v)

    copies = []
    for j in range(NCH):
        copies.append(pltpu.async_copy(
            utab_hbm.at[idx_u.at[j]], rows_u.at[pl.ds(j * CHUNK, CHUNK)],
            sem_u))
        copies.append(pltpu.async_copy(
            itab_hbm.at[idx_i.at[j]], rows_i.at[pl.ds(j * CHUNK, CHUNK)],
            sem_i))
    for c in copies:
        c.wait()

    wu = [w_v[pl.ds(k * LANES, LANES)] for k in range(4)]
    wi = [w_v[pl.ds(D + k * LANES, LANES)] for k in range(4)]
    bias_vec = w_v[pl.ds(2 * D, LANES)]
    rows16 = lax.iota(jnp.int32, LANES)

    def body(g, carry):
        b0 = g * LANES
        for l in range(LANES):
            b = b0 + l
            p0 = rows_u[b, pl.ds(0, LANES)] * wu[0]
            p1 = rows_u[b, pl.ds(LANES, LANES)] * wu[1]
            p2 = rows_u[b, pl.ds(2 * LANES, LANES)] * wu[2]
            p3 = rows_u[b, pl.ds(3 * LANES, LANES)] * wu[3]
            q0 = rows_i[b, pl.ds(0, LANES)] * wi[0]
            q1 = rows_i[b, pl.ds(LANES, LANES)] * wi[1]
            q2 = rows_i[b, pl.ds(2 * LANES, LANES)] * wi[2]
            q3 = rows_i[b, pl.ds(3 * LANES, LANES)] * wi[3]
            tr_v[l, pl.ds(0, LANES)] = (
                ((p0 + p1) + (p2 + p3)) + ((q0 + q1) + (q2 + q3)))
        acc = bias_vec
        for j in range(LANES):
            acc = acc + plsc.load_gather(
                tr_v, [rows16, jnp.full((LANES,), j, jnp.int32)])
        out_v[pl.ds(b0, LANES)] = acc
        return carry

    lax.fori_loop(0, BPW // LANES, body, 0)

    pltpu.sync_copy(out_v, out_hbm.at[pl.ds(base, BPW)])


def kernel(user_ids, item_ids, user_table, item_table, fc_w, fc_b):
    uid = user_ids.astype(jnp.int32).reshape(NW * NCH, CHUNK)
    iid = item_ids.astype(jnp.int32).reshape(NW * NCH, CHUNK)
    wb = jnp.concatenate(
        [fc_w.reshape(-1), jnp.full((LANES,), fc_b[0], jnp.float32)])
    return _rec_kernel(uid, iid, user_table, item_table, wb)


# speedup vs baseline: 1.3181x; 1.3181x over previous
import functools

import jax
import jax.numpy as jnp
from jax import lax
from jax.experimental import pallas as pl
from jax.experimental.pallas import tpu as pltpu
from jax.experimental.pallas import tpu_sc as plsc

NC = 2
NS = 16
LANES = 16
NW = NC * NS
BATCH = 16384
D = 64
BPW = BATCH // NW
NUSER = 1000000
NITEM = 100000
CW = 128

UT_FULL = NUSER // CW
UT_BASE = UT_FULL // NW
UT_EXTRA = UT_FULL - UT_BASE * NW
UTAIL = NUSER - UT_FULL * CW
IT_FULL = NITEM // CW
IT_BASE = IT_FULL // NW
IT_EXTRA = IT_FULL - IT_BASE * NW
ITAIL = NITEM - IT_FULL * CW

_mesh = plsc.VectorSubcoreMesh(
    core_axis_name="c", subcore_axis_name="s", num_cores=NC, num_subcores=NS
)


def _dot_chunk(buf, w_v, w_off, width, out_ref, out_off):
    ngrp = width // LANES
    wvecs = [w_v[pl.ds(w_off + k * LANES, LANES)] for k in range(D // LANES)]
    accs = [None] * ngrp
    for cb in range(D // LANES):
        spl = [lax.broadcast(wvecs[cb][j], (LANES,)) for j in range(LANES)]
        for grp in range(ngrp):
            a = accs[grp]
            for j in range(LANES):
                c = cb * LANES + j
                p = buf[c, pl.ds(grp * LANES, LANES)] * spl[j]
                a = p if a is None else a + p
            accs[grp] = a
    for grp in range(ngrp):
        out_ref[pl.ds(out_off + grp * LANES, LANES)] = accs[grp]


@functools.partial(
    pl.kernel,
    out_type=(
        jax.ShapeDtypeStruct((NUSER,), jnp.float32),
        jax.ShapeDtypeStruct((NITEM,), jnp.float32),
    ),
    mesh=_mesh,
    scratch_types=[
        pltpu.VMEM((136,), jnp.float32),
        pltpu.VMEM((D, CW), jnp.float32),
        pltpu.VMEM((D, CW), jnp.float32),
        pltpu.VMEM(((UT_BASE + 1) * CW,), jnp.float32),
        pltpu.VMEM(((IT_BASE + 1) * CW,), jnp.float32),
        pltpu.VMEM((D, UTAIL), jnp.float32),
        pltpu.VMEM((D, ITAIL), jnp.float32),
        pltpu.VMEM((UTAIL,), jnp.float32),
        pltpu.VMEM((ITAIL,), jnp.float32),
        pltpu.SemaphoreType.DMA,
    ],
)
def _sweep(tu_hbm, ti_hbm, w_hbm, su_hbm, si_hbm,
           w_v, buf_a, buf_b, s_uv, s_iv, tb_u, tb_i, ts_u, ts_i, sem):
    wid = lax.axis_index("s") * NC + lax.axis_index("c")

    pltpu.sync_copy(w_hbm, w_v)

    def sweep_table(t_hbm, w_off, start, total, s_v):
        def fire(t, buf):
            off = pl.multiple_of(t * CW, CW)
            pltpu.async_copy(t_hbm.at[:, pl.ds(off, CW)], buf, sem)

        def drain():
            pltpu.make_async_copy(
                t_hbm.at[:, pl.ds(0, CW)], buf_a, sem).wait()

        fire(start, buf_a)

        def body(g, carry):
            @pl.when(g % 2 == 0)
            def _():
                drain()

                @pl.when(g + 1 < total)
                def _():
                    fire(start + g + 1, buf_b)

                _dot_chunk(buf_a, w_v, w_off, CW, s_v, g * CW)

            @pl.when(g % 2 == 1)
            def _():
                drain()

                @pl.when(g + 1 < total)
                def _():
                    fire(start + g + 1, buf_a)

                _dot_chunk(buf_b, w_v, w_off, CW, s_v, g * CW)

            return carry

        lax.fori_loop(0, total, body, 0)

    ustart = wid * UT_BASE + jnp.minimum(wid, UT_EXTRA)
    utotal = UT_BASE + (wid < UT_EXTRA).astype(jnp.int32)
    sweep_table(tu_hbm, 0, ustart, utotal, s_uv)
    pltpu.sync_copy(s_uv.at[pl.ds(0, UT_BASE * CW)],
                    su_hbm.at[pl.ds(ustart * CW, UT_BASE * CW)])

    @pl.when(wid < UT_EXTRA)
    def _():
        pltpu.sync_copy(
            s_uv.at[pl.ds(UT_BASE * CW, CW)],
            su_hbm.at[pl.ds(ustart * CW + UT_BASE * CW, CW)])

    istart = wid * IT_BASE + jnp.minimum(wid, IT_EXTRA)
    itotal = IT_BASE + (wid < IT_EXTRA).astype(jnp.int32)
    sweep_table(ti_hbm, D, istart, itotal, s_iv)
    pltpu.sync_copy(s_iv.at[pl.ds(0, IT_BASE * CW)],
                    si_hbm.at[pl.ds(istart * CW, IT_BASE * CW)])

    @pl.when(wid < IT_EXTRA)
    def _():
        pltpu.sync_copy(
            s_iv.at[pl.ds(IT_BASE * CW, CW)],
            si_hbm.at[pl.ds(istart * CW + IT_BASE * CW, CW)])

    @pl.when(wid == NW - 1)
    def _():
        pltpu.sync_copy(tu_hbm.at[:, pl.ds(UT_FULL * CW, UTAIL)], tb_u)
        _dot_chunk(tb_u, w_v, 0, UTAIL, ts_u, 0)
        pltpu.sync_copy(ts_u, su_hbm.at[pl.ds(UT_FULL * CW, UTAIL)])
        pltpu.sync_copy(ti_hbm.at[:, pl.ds(IT_FULL * CW, ITAIL)], tb_i)
        _dot_chunk(tb_i, w_v, D, ITAIL, ts_i, 0)
        pltpu.sync_copy(ts_i, si_hbm.at[pl.ds(IT_FULL * CW, ITAIL)])


@functools.partial(
    pl.kernel,
    out_type=jax.ShapeDtypeStruct((BATCH,), jnp.float32),
    mesh=_mesh,
    scratch_types=[
        pltpu.VMEM((4, 128), jnp.int32),
        pltpu.VMEM((4, 128), jnp.int32),
        pltpu.VMEM((BPW,), jnp.float32),
        pltpu.VMEM((BPW,), jnp.float32),
        pltpu.VMEM((LANES,), jnp.float32),
        pltpu.VMEM((BPW,), jnp.float32),
        pltpu.SemaphoreType.DMA,
        pltpu.SemaphoreType.DMA,
    ],
    compiler_params=pltpu.CompilerParams(
        needs_layout_passes=False, use_tc_tiling_on_sc=False),
)
def _gather_out(su_hbm, si_hbm, uid_hbm, iid_hbm, bv_hbm, out_hbm,
                idx_u, idx_i, g_u, g_i, b_v, out_v, sem_u, sem_i):
    wid = lax.axis_index("s") * NC + lax.axis_index("c")
    base = wid * BPW

    pltpu.sync_copy(uid_hbm.at[pl.ds(wid * 4, 4)], idx_u)
    pltpu.sync_copy(iid_hbm.at[pl.ds(wid * 4, 4)], idx_i)
    pltpu.sync_copy(bv_hbm, b_v)

    copies = []
    for j in range(4):
        copies.append(pltpu.async_copy(
            su_hbm.at[idx_u.at[j]], g_u.at[pl.ds(j * 128, 128)], sem_u))
        copies.append(pltpu.async_copy(
            si_hbm.at[idx_i.at[j]], g_i.at[pl.ds(j * 128, 128)], sem_i))
    for c in copies:
        c.wait()

    bias = b_v[...]

    def body(g, carry):
        b0 = g * LANES
        out_v[pl.ds(b0, LANES)] = (
            g_u[pl.ds(b0, LANES)] + g_i[pl.ds(b0, LANES)] + bias)
        return carry

    lax.fori_loop(0, BPW // LANES, body, 0)

    pltpu.sync_copy(out_v, out_hbm.at[pl.ds(base, BPW)])


def kernel(user_ids, item_ids, user_table, item_table, fc_w, fc_b):
    t_u = user_table.T
    t_i = item_table.T
    w_pad = jnp.concatenate(
        [fc_w.reshape(-1), jnp.zeros((8,), jnp.float32)])
    s_u, s_i = _sweep(t_u, t_i, w_pad)
    uid = user_ids.astype(jnp.int32).reshape(NW * 4, 128)
    iid = item_ids.astype(jnp.int32).reshape(NW * 4, 128)
    bv = jnp.full((LANES,), fc_b[0], jnp.float32)
    return _gather_out(s_u, s_i, uid, iid, bv)


# speedup vs baseline: 4.6019x; 3.4912x over previous
import functools

import jax
import jax.numpy as jnp
from jax import lax
from jax.experimental import pallas as pl
from jax.experimental.pallas import tpu as pltpu
from jax.experimental.pallas import tpu_sc as plsc

NC = 2
NS = 16
LANES = 16
NW = NC * NS
BATCH = 16384
D = 64
BPW = BATCH // NW
NUSER = 1000000
NITEM = 100000
CW = 128
NBUF = 6

UT_FULL = NUSER // CW
UT_BASE = UT_FULL // NW
UT_EXTRA = UT_FULL - UT_BASE * NW
UTAIL = NUSER - UT_FULL * CW
IT_FULL = NITEM // CW
IT_BASE = IT_FULL // NW
IT_EXTRA = IT_FULL - IT_BASE * NW
ITAIL = NITEM - IT_FULL * CW

_mesh = plsc.VectorSubcoreMesh(
    core_axis_name="c", subcore_axis_name="s", num_cores=NC, num_subcores=NS
)


def _dot_chunk(buf, row0, w_v, w_off, width, out_ref, out_off):
    ngrp = width // LANES
    gblk = min(4, ngrp)
    wvecs = [w_v[pl.ds(w_off + k * LANES, LANES)] for k in range(D // LANES)]
    for gb0 in range(0, ngrp, gblk):
        nb = min(gblk, ngrp - gb0)
        accs = [None] * nb
        for cb in range(D // LANES):
            spl = [lax.broadcast(wvecs[cb][j], (LANES,)) for j in range(LANES)]
            for g in range(nb):
                grp = gb0 + g
                a = accs[g]
                for j in range(LANES):
                    c = cb * LANES + j
                    p = buf[row0 + c, pl.ds(grp * LANES, LANES)] * spl[j]
                    a = p if a is None else a + p
                accs[g] = a
        for g in range(nb):
            out_ref[pl.ds(out_off + (gb0 + g) * LANES, LANES)] = accs[g]


@functools.partial(
    pl.kernel,
    out_type=(
        jax.ShapeDtypeStruct((NUSER,), jnp.float32),
        jax.ShapeDtypeStruct((NITEM,), jnp.float32),
    ),
    mesh=_mesh,
    scratch_types=[
        pltpu.VMEM((136,), jnp.float32),
        pltpu.VMEM((NBUF * D, CW), jnp.float32),
        pltpu.VMEM(((UT_BASE + 1) * CW,), jnp.float32),
        pltpu.VMEM(((IT_BASE + 1) * CW,), jnp.float32),
        pltpu.VMEM((D, UTAIL), jnp.float32),
        pltpu.VMEM((D, ITAIL), jnp.float32),
        pltpu.VMEM((UTAIL,), jnp.float32),
        pltpu.VMEM((ITAIL,), jnp.float32),
        pltpu.SemaphoreType.DMA,
    ],
)
def _sweep(tu_hbm, ti_hbm, w_hbm, su_hbm, si_hbm,
           w_v, ring, s_uv, s_iv, tb_u, tb_i, ts_u, ts_i, sem):
    wid = lax.axis_index("s") * NC + lax.axis_index("c")

    pltpu.sync_copy(w_hbm, w_v)

    def sweep_table(t_hbm, w_off, start, total, s_v):
        def fire(t, slot):
            off = pl.multiple_of(t * CW, CW)
            row = pl.multiple_of(slot * D, D)
            pltpu.async_copy(
                t_hbm.at[:, pl.ds(off, CW)], ring.at[pl.ds(row, D)], sem)

        def drain():
            pltpu.make_async_copy(
                t_hbm.at[:, pl.ds(0, CW)], ring.at[pl.ds(0, D)], sem).wait()

        for k in range(NBUF):
            @pl.when(k < total)
            def _(k=k):
                fire(start + k, k)

        def body(t, carry):
            slot = lax.rem(t, NBUF)
            row0 = pl.multiple_of(slot * D, D)
            drain()
            _dot_chunk(ring, row0, w_v, w_off, CW, s_v, t * CW)

            @pl.when(t + NBUF < total)
            def _():
                fire(start + t + NBUF, slot)

            return carry

        lax.fori_loop(0, total, body, 0)

    ustart = wid * UT_BASE + jnp.minimum(wid, UT_EXTRA)
    utotal = UT_BASE + (wid < UT_EXTRA).astype(jnp.int32)
    sweep_table(tu_hbm, 0, ustart, utotal, s_uv)
    pltpu.sync_copy(s_uv.at[pl.ds(0, UT_BASE * CW)],
                    su_hbm.at[pl.ds(ustart * CW, UT_BASE * CW)])

    @pl.when(wid < UT_EXTRA)
    def _():
        pltpu.sync_copy(
            s_uv.at[pl.ds(UT_BASE * CW, CW)],
            su_hbm.at[pl.ds(ustart * CW + UT_BASE * CW, CW)])

    istart = wid * IT_BASE + jnp.minimum(wid, IT_EXTRA)
    itotal = IT_BASE + (wid < IT_EXTRA).astype(jnp.int32)
    sweep_table(ti_hbm, D, istart, itotal, s_iv)
    pltpu.sync_copy(s_iv.at[pl.ds(0, IT_BASE * CW)],
                    si_hbm.at[pl.ds(istart * CW, IT_BASE * CW)])

    @pl.when(wid < IT_EXTRA)
    def _():
        pltpu.sync_copy(
            s_iv.at[pl.ds(IT_BASE * CW, CW)],
            si_hbm.at[pl.ds(istart * CW + IT_BASE * CW, CW)])

    @pl.when(wid == NW - 1)
    def _():
        pltpu.sync_copy(tu_hbm.at[:, pl.ds(UT_FULL * CW, UTAIL)], tb_u)
        _dot_chunk(tb_u, 0, w_v, 0, UTAIL, ts_u, 0)
        pltpu.sync_copy(ts_u, su_hbm.at[pl.ds(UT_FULL * CW, UTAIL)])
        pltpu.sync_copy(ti_hbm.at[:, pl.ds(IT_FULL * CW, ITAIL)], tb_i)
        _dot_chunk(tb_i, 0, w_v, D, ITAIL, ts_i, 0)
        pltpu.sync_copy(ts_i, si_hbm.at[pl.ds(IT_FULL * CW, ITAIL)])


@functools.partial(
    pl.kernel,
    out_type=jax.ShapeDtypeStruct((BATCH,), jnp.float32),
    mesh=_mesh,
    scratch_types=[
        pltpu.VMEM((4, 128), jnp.int32),
        pltpu.VMEM((4, 128), jnp.int32),
        pltpu.VMEM((BPW,), jnp.float32),
        pltpu.VMEM((BPW,), jnp.float32),
        pltpu.VMEM((LANES,), jnp.float32),
        pltpu.VMEM((BPW,), jnp.float32),
        pltpu.SemaphoreType.DMA,
        pltpu.SemaphoreType.DMA,
    ],
    compiler_params=pltpu.CompilerParams(
        needs_layout_passes=False, use_tc_tiling_on_sc=False),
)
def _gather_out(su_hbm, si_hbm, uid_hbm, iid_hbm, bv_hbm, out_hbm,
                idx_u, idx_i, g_u, g_i, b_v, out_v, sem_u, sem_i):
    wid = lax.axis_index("s") * NC + lax.axis_index("c")
    base = wid * BPW

    pltpu.sync_copy(uid_hbm.at[pl.ds(wid * 4, 4)], idx_u)
    pltpu.sync_copy(iid_hbm.at[pl.ds(wid * 4, 4)], idx_i)
    pltpu.sync_copy(bv_hbm, b_v)

    copies = []
    for j in range(4):
        copies.append(pltpu.async_copy(
            su_hbm.at[idx_u.at[j]], g_u.at[pl.ds(j * 128, 128)], sem_u))
        copies.append(pltpu.async_copy(
            si_hbm.at[idx_i.at[j]], g_i.at[pl.ds(j * 128, 128)], sem_i))
    for c in copies:
        c.wait()

    bias = b_v[...]

    def body(g, carry):
        b0 = g * LANES
        out_v[pl.ds(b0, LANES)] = (
            g_u[pl.ds(b0, LANES)] + g_i[pl.ds(b0, LANES)] + bias)
        return carry

    lax.fori_loop(0, BPW // LANES, body, 0)

    pltpu.sync_copy(out_v, out_hbm.at[pl.ds(base, BPW)])


def kernel(user_ids, item_ids, user_table, item_table, fc_w, fc_b):
    t_u = user_table.T
    t_i = item_table.T
    w_pad = jnp.concatenate(
        [fc_w.reshape(-1), jnp.zeros((8,), jnp.float32)])
    s_u, s_i = _sweep(t_u, t_i, w_pad)
    uid = user_ids.astype(jnp.int32).reshape(NW * 4, 128)
    iid = item_ids.astype(jnp.int32).reshape(NW * 4, 128)
    bv = jnp.full((LANES,), fc_b[0], jnp.float32)
    return _gather_out(s_u, s_i, uid, iid, bv)


# speedup vs baseline: 4.6106x; 1.0019x over previous
import functools

import jax
import jax.numpy as jnp
from jax import lax
from jax.experimental import pallas as pl
from jax.experimental.pallas import tpu as pltpu
from jax.experimental.pallas import tpu_sc as plsc

NC = 2
NS = 16
LANES = 16
NW = NC * NS
BATCH = 16384
D = 64
BPW = BATCH // NW
NUSER = 1000000
NITEM = 100000
CW = 128
NBUF = 8

UT_FULL = NUSER // CW
UT_BASE = UT_FULL // NW
UT_EXTRA = UT_FULL - UT_BASE * NW
UTAIL = NUSER - UT_FULL * CW
IT_FULL = NITEM // CW
IT_BASE = IT_FULL // NW
IT_EXTRA = IT_FULL - IT_BASE * NW
ITAIL = NITEM - IT_FULL * CW

_mesh = plsc.VectorSubcoreMesh(
    core_axis_name="c", subcore_axis_name="s", num_cores=NC, num_subcores=NS
)


def _dot_chunk(buf, row0, w_v, w_off, width, out_ref, out_off):
    ngrp = width // LANES
    gblk = min(4, ngrp)
    wvecs = [w_v[pl.ds(w_off + k * LANES, LANES)] for k in range(D // LANES)]
    for gb0 in range(0, ngrp, gblk):
        nb = min(gblk, ngrp - gb0)
        accs = [None] * nb
        for cb in range(D // LANES):
            spl = [lax.broadcast(wvecs[cb][j], (LANES,)) for j in range(LANES)]
            for g in range(nb):
                grp = gb0 + g
                a = accs[g]
                for j in range(LANES):
                    c = cb * LANES + j
                    p = buf[row0 + c, pl.ds(grp * LANES, LANES)] * spl[j]
                    a = p if a is None else a + p
                accs[g] = a
        for g in range(nb):
            out_ref[pl.ds(out_off + (gb0 + g) * LANES, LANES)] = accs[g]


@functools.partial(
    pl.kernel,
    out_type=(
        jax.ShapeDtypeStruct((NUSER,), jnp.float32),
        jax.ShapeDtypeStruct((NITEM,), jnp.float32),
    ),
    mesh=_mesh,
    scratch_types=[
        pltpu.VMEM((136,), jnp.float32),
        pltpu.VMEM((NBUF * D, CW), jnp.float32),
        pltpu.VMEM(((UT_BASE + 1) * CW,), jnp.float32),
        pltpu.VMEM(((IT_BASE + 1) * CW,), jnp.float32),
        pltpu.VMEM((D, UTAIL), jnp.float32),
        pltpu.VMEM((D, ITAIL), jnp.float32),
        pltpu.VMEM((UTAIL,), jnp.float32),
        pltpu.VMEM((ITAIL,), jnp.float32),
        pltpu.SemaphoreType.DMA,
    ],
)
def _sweep(tu_hbm, ti_hbm, w_hbm, su_hbm, si_hbm,
           w_v, ring, s_uv, s_iv, tb_u, tb_i, ts_u, ts_i, sem):
    wid = lax.axis_index("s") * NC + lax.axis_index("c")

    pltpu.sync_copy(w_hbm, w_v)

    def sweep_table(t_hbm, w_off, start, total, s_v):
        def fire(t, slot):
            off = pl.multiple_of(t * CW, CW)
            row = pl.multiple_of(slot * D, D)
            pltpu.async_copy(
                t_hbm.at[:, pl.ds(off, CW)], ring.at[pl.ds(row, D)], sem)

        def drain():
            pltpu.make_async_copy(
                t_hbm.at[:, pl.ds(0, CW)], ring.at[pl.ds(0, D)], sem).wait()

        for k in range(NBUF):
            @pl.when(k < total)
            def _(k=k):
                fire(start + k, k)

        def body(t, carry):
            slot = lax.rem(t, NBUF)
            row0 = pl.multiple_of(slot * D, D)
            drain()
            _dot_chunk(ring, row0, w_v, w_off, CW, s_v, t * CW)

            @pl.when(t + NBUF < total)
            def _():
                fire(start + t + NBUF, slot)

            return carry

        lax.fori_loop(0, total, body, 0)

    ustart = wid * UT_BASE + jnp.minimum(wid, UT_EXTRA)
    utotal = UT_BASE + (wid < UT_EXTRA).astype(jnp.int32)
    sweep_table(tu_hbm, 0, ustart, utotal, s_uv)
    pltpu.sync_copy(s_uv.at[pl.ds(0, UT_BASE * CW)],
                    su_hbm.at[pl.ds(ustart * CW, UT_BASE * CW)])

    @pl.when(wid < UT_EXTRA)
    def _():
        pltpu.sync_copy(
            s_uv.at[pl.ds(UT_BASE * CW, CW)],
            su_hbm.at[pl.ds(ustart * CW + UT_BASE * CW, CW)])

    istart = wid * IT_BASE + jnp.minimum(wid, IT_EXTRA)
    itotal = IT_BASE + (wid < IT_EXTRA).astype(jnp.int32)
    sweep_table(ti_hbm, D, istart, itotal, s_iv)
    pltpu.sync_copy(s_iv.at[pl.ds(0, IT_BASE * CW)],
                    si_hbm.at[pl.ds(istart * CW, IT_BASE * CW)])

    @pl.when(wid < IT_EXTRA)
    def _():
        pltpu.sync_copy(
            s_iv.at[pl.ds(IT_BASE * CW, CW)],
            si_hbm.at[pl.ds(istart * CW + IT_BASE * CW, CW)])

    @pl.when(wid == NW - 1)
    def _():
        pltpu.sync_copy(tu_hbm.at[:, pl.ds(UT_FULL * CW, UTAIL)], tb_u)
        _dot_chunk(tb_u, 0, w_v, 0, UTAIL, ts_u, 0)
        pltpu.sync_copy(ts_u, su_hbm.at[pl.ds(UT_FULL * CW, UTAIL)])
        pltpu.sync_copy(ti_hbm.at[:, pl.ds(IT_FULL * CW, ITAIL)], tb_i)
        _dot_chunk(tb_i, 0, w_v, D, ITAIL, ts_i, 0)
        pltpu.sync_copy(ts_i, si_hbm.at[pl.ds(IT_FULL * CW, ITAIL)])


@functools.partial(
    pl.kernel,
    out_type=jax.ShapeDtypeStruct((BATCH,), jnp.float32),
    mesh=_mesh,
    scratch_types=[
        pltpu.VMEM((4, 128), jnp.int32),
        pltpu.VMEM((4, 128), jnp.int32),
        pltpu.VMEM((BPW,), jnp.float32),
        pltpu.VMEM((BPW,), jnp.float32),
        pltpu.VMEM((LANES,), jnp.float32),
        pltpu.VMEM((BPW,), jnp.float32),
        pltpu.SemaphoreType.DMA,
        pltpu.SemaphoreType.DMA,
    ],
    compiler_params=pltpu.CompilerParams(
        needs_layout_passes=False, use_tc_tiling_on_sc=False),
)
def _gather_out(su_hbm, si_hbm, uid_hbm, iid_hbm, bv_hbm, out_hbm,
                idx_u, idx_i, g_u, g_i, b_v, out_v, sem_u, sem_i):
    wid = lax.axis_index("s") * NC + lax.axis_index("c")
    base = wid * BPW

    pltpu.sync_copy(uid_hbm.at[pl.ds(wid * 4, 4)], idx_u)
    pltpu.sync_copy(iid_hbm.at[pl.ds(wid * 4, 4)], idx_i)
    pltpu.sync_copy(bv_hbm, b_v)

    copies = []
    for j in range(4):
        copies.append(pltpu.async_copy(
            su_hbm.at[idx_u.at[j]], g_u.at[pl.ds(j * 128, 128)], sem_u))
        copies.append(pltpu.async_copy(
            si_hbm.at[idx_i.at[j]], g_i.at[pl.ds(j * 128, 128)], sem_i))
    for c in copies:
        c.wait()

    bias = b_v[...]

    def body(g, carry):
        b0 = g * LANES
        out_v[pl.ds(b0, LANES)] = (
            g_u[pl.ds(b0, LANES)] + g_i[pl.ds(b0, LANES)] + bias)
        return carry

    lax.fori_loop(0, BPW // LANES, body, 0)

    pltpu.sync_copy(out_v, out_hbm.at[pl.ds(base, BPW)])


def kernel(user_ids, item_ids, user_table, item_table, fc_w, fc_b):
    t_u = user_table.T
    t_i = item_table.T
    w_pad = jnp.concatenate(
        [fc_w.reshape(-1), jnp.zeros((8,), jnp.float32)])
    s_u, s_i = _sweep(t_u, t_i, w_pad)
    uid = user_ids.astype(jnp.int32).reshape(NW * 4, 128)
    iid = item_ids.astype(jnp.int32).reshape(NW * 4, 128)
    bv = jnp.full((LANES,), fc_b[0], jnp.float32)
    return _gather_out(s_u, s_i, uid, iid, bv)
